# baseline (speedup 1.0000x reference)
import math

import jax
import jax.numpy as jnp
from jax import lax
from jax.experimental import pallas as pl
from jax.experimental.pallas import tpu as pltpu
from jax.experimental.pallas import tpu_sc as plsc

_N = 20000
_NODES = 256
_NEIGH = 16
_VPTS = 3
_THRESH = math.cos(math.radians(5.0))
_CHUNK = 64
_TILES_USED = (_VPTS * _NODES) // _CHUNK


def _nms_body(st_ref, idx8_ref, idx1_ref, vp_ref, x1_ref, y1_ref, z1_ref):
    s = st_ref[0:1, :]
    x = st_ref[1:2, :]
    y = st_ref[2:3, :]
    z = st_ref[3:4, :]
    n = jnp.sqrt(x * x + y * y + z * z) + 1e-12
    ux, uy, uz = x / n, y / n, z / n
    x1_ref[...] = x.reshape(_N)
    y1_ref[...] = y.reshape(_N)
    z1_ref[...] = z.reshape(_N)

    flat = lax.broadcasted_iota(jnp.int32, (1, _N), 1)

    picked = []
    uvecs = []
    for _ in range(_VPTS):
        m = jnp.max(s)
        i = jnp.min(jnp.where(s == m, flat, jnp.int32(2**31 - 1)))
        picked.append(i)
        sel = flat == i
        xi = jnp.sum(jnp.where(sel, ux, 0.0))
        yi = jnp.sum(jnp.where(sel, uy, 0.0))
        zi = jnp.sum(jnp.where(sel, uz, 0.0))
        uvecs.append((xi, yi, zi))
        sim = jnp.abs(ux * xi + uy * yi + uz * zi)
        s = jnp.where(sim > _THRESH, -jnp.inf, s)

    lane8 = lax.broadcasted_iota(jnp.int32, (1, 8), 1)
    out = jnp.where(lane8 == 1, picked[1], picked[0])
    out = jnp.where(lane8 == 2, picked[2], out)
    idx8_ref[...] = out
    idx1_ref[...] = out[:, :_VPTS]

    lane3 = lax.broadcasted_iota(jnp.int32, (1, _VPTS), 1)
    for k in range(_VPTS):
        xi, yi, zi = uvecs[k]
        vp_ref[k:k + 1, :] = jnp.where(
            lane3 == 0, xi, jnp.where(lane3 == 1, yi, zi))


def _sc_gather_body(idx8_hbm, inds_hbm, x1_hbm, y1_hbm, z1_hbm, s1_hbm,
                    ind1_out, xo_hbm, yo_hbm, zo_hbm, so_hbm,
                    idx8_v, rows8_v, idxc_v, g_v, sem, wsem):
    wid = lax.axis_index("s")

    @pl.when(wid < _TILES_USED)
    def _():
        pltpu.sync_copy(idx8_hbm, idx8_v)
        pltpu.async_copy(inds_hbm.at[idx8_v.at[0]], rows8_v, sem).wait()

        @pl.when(wid == 0)
        def _():
            pltpu.sync_copy(rows8_v.at[pl.ds(0, _VPTS)], ind1_out)

        vpt = wid // (_NODES // _CHUNK)
        c0 = (wid % (_NODES // _CHUNK)) * _CHUNK
        for j in range(_CHUNK // 16):
            idxc_v[pl.ds(j * 16, 16)] = rows8_v[vpt, pl.ds(c0 + j * 16, 16)]
        copies = [
            pltpu.async_copy(tab.at[idxc_v], g_v.at[c], sem)
            for c, tab in enumerate((x1_hbm, y1_hbm, z1_hbm, s1_hbm))
        ]
        for cp in copies:
            cp.wait()
        writes = [
            pltpu.async_copy(g_v.at[c], out.at[pl.ds(wid * _CHUNK, _CHUNK)],
                             wsem)
            for c, out in enumerate((xo_hbm, yo_hbm, zo_hbm, so_hbm))
        ]
        for w in writes:
            w.wait()


def _graph_body(x_ref, y_ref, z_ref, s_ref, pred_ref, edge_ref):
    coli = lax.broadcasted_iota(jnp.int32, (_NODES, _NODES), 1)
    rowi = lax.broadcasted_iota(jnp.int32, (_NODES, _NODES), 0)
    colf = coli.astype(jnp.float32)
    iden = (rowi == coli).astype(jnp.float32)
    xa = x_ref[...].reshape(1, _VPTS * _NODES)
    ya = y_ref[...].reshape(1, _VPTS * _NODES)
    za = z_ref[...].reshape(1, _VPTS * _NODES)
    sa = s_ref[...].reshape(1, _VPTS * _NODES)
    for v in range(_VPTS):
        sl = slice(v * _NODES, (v + 1) * _NODES)
        x, y, z, s = xa[:, sl], ya[:, sl], za[:, sl], sa[:, sl]
        n = jnp.sqrt(x * x + y * y + z * z) + 1e-12
        u = jnp.concatenate([x / n, y / n, z / n], axis=0)
        dis = lax.dot_general(u, u, (((0,), (0,)), ((), ())),
                              preferred_element_type=jnp.float32)
        a = jnp.abs(dis)
        sc = lax.dot_general(iden, s, (((1,), (1,)), ((), ())),
                             preferred_element_type=jnp.float32)
        vals_cols = []
        nb_cols = []
        for _ in range(_NEIGH):
            m = jnp.max(a, axis=1, keepdims=True)
            nbf = jnp.min(jnp.where(a == m, colf, jnp.float32(_NODES)),
                          axis=1, keepdims=True)
            vals_cols.append(m)
            nb_cols.append(nbf)
            a = jnp.where(colf == nbf, -jnp.inf, a)
        vals = jnp.concatenate(vals_cols, axis=1)
        nbs = jnp.concatenate(nb_cols, axis=1).astype(jnp.int32)
        pred_ref[v] = vals + 0.1 * sc
        edge_ref[v] = nbs


def kernel(scores, xyz, ind1_scale):
    f32 = jnp.float32
    i32 = jnp.int32

    st = jnp.concatenate([scores[None, :], xyz.T], axis=0)
    idx8, idx1, vpts1, x1d, y1d, z1d = pl.pallas_call(
        _nms_body,
        out_shape=(jax.ShapeDtypeStruct((1, 8), i32),
                   jax.ShapeDtypeStruct((1, _VPTS), i32),
                   jax.ShapeDtypeStruct((_VPTS, _VPTS), f32),
                   jax.ShapeDtypeStruct((_N,), f32),
                   jax.ShapeDtypeStruct((_N,), f32),
                   jax.ShapeDtypeStruct((_N,), f32)),
    )(st)

    g768 = jax.ShapeDtypeStruct((_VPTS * _NODES,), f32)
    sc_fn = pl.kernel(
        _sc_gather_body,
        out_type=(jax.ShapeDtypeStruct((_VPTS, _NODES), i32),
                  g768, g768, g768, g768),
        mesh=plsc.VectorSubcoreMesh(core_axis_name="c", subcore_axis_name="s",
                                    num_cores=1),
        scratch_types=[
            pltpu.VMEM((1, 8), i32),
            pltpu.VMEM((8, _NODES), i32),
            pltpu.VMEM((_CHUNK,), i32),
            pltpu.VMEM((4, _CHUNK), f32),
            pltpu.SemaphoreType.DMA,
            pltpu.SemaphoreType.DMA,
        ],
    )
    ind1, xo, yo, zo, so = sc_fn(idx8, ind1_scale, x1d, y1d, z1d, scores)

    pred, nbs = pl.pallas_call(
        _graph_body,
        out_shape=(jax.ShapeDtypeStruct((_VPTS, _NODES, _NEIGH), f32),
                   jax.ShapeDtypeStruct((_VPTS, _NODES, _NEIGH), i32)),
    )(xo, yo, zo, so)

    center = jnp.broadcast_to(
        jnp.repeat(jnp.arange(_NODES, dtype=i32), _NEIGH)[None, :],
        (_VPTS, _NODES * _NEIGH))
    edge1 = jnp.stack([center, nbs.reshape(_VPTS, -1)], axis=1)
    return pred, idx1.reshape(_VPTS), ind1, edge1, vpts1


# speedup vs baseline: 1.0071x; 1.0071x over previous
import math

import jax
import jax.numpy as jnp
from jax import lax
from jax.experimental import pallas as pl
from jax.experimental.pallas import tpu as pltpu
from jax.experimental.pallas import tpu_sc as plsc

_N = 20000
_NODES = 256
_NEIGH = 16
_VPTS = 3
_THRESH = math.cos(math.radians(5.0))
_CHUNK = 64
_TILES_USED = (_VPTS * _NODES) // _CHUNK


_SEG = 20480


def _nms_body(st_ref, idx8_ref, idx1_ref, vp_ref, xf_ref):
    s = st_ref[0:1, :]
    x = st_ref[1:2, :]
    y = st_ref[2:3, :]
    z = st_ref[3:4, :]
    n = jnp.sqrt(x * x + y * y + z * z) + 1e-12
    ux, uy, uz = x / n, y / n, z / n
    xf_ref[pl.ds(0 * _SEG, _N)] = x.reshape(_N)
    xf_ref[pl.ds(1 * _SEG, _N)] = y.reshape(_N)
    xf_ref[pl.ds(2 * _SEG, _N)] = z.reshape(_N)

    flat = lax.broadcasted_iota(jnp.int32, (1, _N), 1)

    picked = []
    uvecs = []
    for _ in range(_VPTS):
        m = jnp.max(s)
        i = jnp.min(jnp.where(s == m, flat, jnp.int32(2**31 - 1)))
        picked.append(i)
        sel = flat == i
        xi = jnp.sum(jnp.where(sel, ux, 0.0))
        yi = jnp.sum(jnp.where(sel, uy, 0.0))
        zi = jnp.sum(jnp.where(sel, uz, 0.0))
        uvecs.append((xi, yi, zi))
        sim = jnp.abs(ux * xi + uy * yi + uz * zi)
        s = jnp.where(sim > _THRESH, -jnp.inf, s)

    lane8 = lax.broadcasted_iota(jnp.int32, (1, 8), 1)
    out = jnp.where(lane8 == 1, picked[1], picked[0])
    out = jnp.where(lane8 == 2, picked[2], out)
    idx8_ref[...] = out
    idx1_ref[...] = out[:, :_VPTS]

    lane3 = lax.broadcasted_iota(jnp.int32, (1, _VPTS), 1)
    for k in range(_VPTS):
        xi, yi, zi = uvecs[k]
        vp_ref[k:k + 1, :] = jnp.where(
            lane3 == 0, xi, jnp.where(lane3 == 1, yi, zi))


def _sc_gather_body(idx8_hbm, inds_hbm, xf_hbm, s1_hbm,
                    ind1_out, go_hbm,
                    idx8_v, rows8_v, idxc4_v, g_v, sem, wsem):
    wid = lax.axis_index("s")

    @pl.when(wid < _TILES_USED)
    def _():
        pltpu.sync_copy(idx8_hbm, idx8_v)
        pltpu.async_copy(inds_hbm.at[idx8_v.at[0]], rows8_v, sem).wait()

        @pl.when(wid == 0)
        def _():
            pltpu.sync_copy(rows8_v.at[pl.ds(0, _VPTS)], ind1_out)

        vpt = wid // (_NODES // _CHUNK)
        c0 = (wid % (_NODES // _CHUNK)) * _CHUNK
        for j in range(_CHUNK // 16):
            base = rows8_v[vpt, pl.ds(c0 + j * 16, 16)]
            idxc4_v[0, pl.ds(j * 16, 16)] = base
            idxc4_v[1, pl.ds(j * 16, 16)] = base + _SEG
            idxc4_v[2, pl.ds(j * 16, 16)] = base + 2 * _SEG
            idxc4_v[3, pl.ds(j * 16, 16)] = base
        copies = [
            pltpu.async_copy(xf_hbm.at[idxc4_v.at[c]], g_v.at[c], sem)
            for c in range(3)
        ]
        copies.append(
            pltpu.async_copy(s1_hbm.at[idxc4_v.at[3]], g_v.at[3], sem))
        for cp in copies:
            cp.wait()
        writes = [
            pltpu.async_copy(
                g_v.at[c],
                go_hbm.at[pl.ds(c * (_VPTS * _NODES) + wid * _CHUNK, _CHUNK)],
                wsem)
            for c in range(4)
        ]
        for w in writes:
            w.wait()


def _graph_body(g_ref, pred_ref, edge_ref):
    coli = lax.broadcasted_iota(jnp.int32, (_NODES, _NODES), 1)
    rowi = lax.broadcasted_iota(jnp.int32, (_NODES, _NODES), 0)
    colf = coli.astype(jnp.float32)
    iden = (rowi == coli).astype(jnp.float32)
    ga = g_ref[...].reshape(1, 4 * _VPTS * _NODES)
    for v in range(_VPTS):
        def comp(c):
            base = c * _VPTS * _NODES + v * _NODES
            return ga[:, base:base + _NODES]
        x, y, z, s = comp(0), comp(1), comp(2), comp(3)
        n = jnp.sqrt(x * x + y * y + z * z) + 1e-12
        u = jnp.concatenate([x / n, y / n, z / n], axis=0)
        dis = lax.dot_general(u, u, (((0,), (0,)), ((), ())),
                              preferred_element_type=jnp.float32)
        a = jnp.abs(dis)
        sc = lax.dot_general(iden, s, (((1,), (1,)), ((), ())),
                             preferred_element_type=jnp.float32)
        vals_cols = []
        nb_cols = []
        for _ in range(_NEIGH):
            m = jnp.max(a, axis=1, keepdims=True)
            nbf = jnp.min(jnp.where(a == m, colf, jnp.float32(_NODES)),
                          axis=1, keepdims=True)
            vals_cols.append(m)
            nb_cols.append(nbf)
            a = jnp.where(colf == nbf, -jnp.inf, a)
        vals = jnp.concatenate(vals_cols, axis=1)
        nbs = jnp.concatenate(nb_cols, axis=1).astype(jnp.int32)
        pred_ref[v] = vals + 0.1 * sc
        edge_ref[v] = nbs


def kernel(scores, xyz, ind1_scale):
    f32 = jnp.float32
    i32 = jnp.int32

    st = jnp.concatenate([scores[None, :], xyz.T], axis=0)
    idx8, idx1, vpts1, xf = pl.pallas_call(
        _nms_body,
        out_shape=(jax.ShapeDtypeStruct((1, 8), i32),
                   jax.ShapeDtypeStruct((1, _VPTS), i32),
                   jax.ShapeDtypeStruct((_VPTS, _VPTS), f32),
                   jax.ShapeDtypeStruct((3 * _SEG,), f32)),
    )(st)

    sc_fn = pl.kernel(
        _sc_gather_body,
        out_type=(jax.ShapeDtypeStruct((_VPTS, _NODES), i32),
                  jax.ShapeDtypeStruct((4 * _VPTS * _NODES,), f32)),
        mesh=plsc.VectorSubcoreMesh(core_axis_name="c", subcore_axis_name="s",
                                    num_cores=1),
        scratch_types=[
            pltpu.VMEM((1, 8), i32),
            pltpu.VMEM((8, _NODES), i32),
            pltpu.VMEM((4, _CHUNK), i32),
            pltpu.VMEM((4, _CHUNK), f32),
            pltpu.SemaphoreType.DMA,
            pltpu.SemaphoreType.DMA,
        ],
    )
    ind1, go = sc_fn(idx8, ind1_scale, xf, scores)

    pred, nbs = pl.pallas_call(
        _graph_body,
        out_shape=(jax.ShapeDtypeStruct((_VPTS, _NODES, _NEIGH), f32),
                   jax.ShapeDtypeStruct((_VPTS, _NODES, _NEIGH), i32)),
    )(go)

    center = jnp.broadcast_to(
        jnp.repeat(jnp.arange(_NODES, dtype=i32), _NEIGH)[None, :],
        (_VPTS, _NODES * _NEIGH))
    edge1 = jnp.stack([center, nbs.reshape(_VPTS, -1)], axis=1)
    return pred, idx1.reshape(_VPTS), ind1, edge1, vpts1


# speedup vs baseline: 1.0485x; 1.0412x over previous
import math

import jax
import jax.numpy as jnp
from jax import lax
from jax.experimental import pallas as pl
from jax.experimental.pallas import tpu as pltpu
from jax.experimental.pallas import tpu_sc as plsc

_N = 20000
_NODES = 256
_NEIGH = 16
_VPTS = 3
_THRESH = math.cos(math.radians(5.0))
_CHUNK = 64
_TILES_USED = (_VPTS * _NODES) // _CHUNK


_SEG = 20480


def _nms_body(st_ref, idx8_ref, idx1_ref, vp_ref, xf_ref):
    s = st_ref[0:1, :]
    x = st_ref[1:2, :]
    y = st_ref[2:3, :]
    z = st_ref[3:4, :]
    n = jnp.sqrt(x * x + y * y + z * z) + 1e-12
    ux, uy, uz = x / n, y / n, z / n
    xf_ref[pl.ds(0 * _SEG, _N)] = x.reshape(_N)
    xf_ref[pl.ds(1 * _SEG, _N)] = y.reshape(_N)
    xf_ref[pl.ds(2 * _SEG, _N)] = z.reshape(_N)

    flat = lax.broadcasted_iota(jnp.int32, (1, _N), 1)

    picked = []
    uvecs = []
    for _ in range(_VPTS):
        m = jnp.max(s)
        i = jnp.min(jnp.where(s == m, flat, jnp.int32(2**31 - 1)))
        picked.append(i)
        sel = flat == i
        xi = jnp.sum(jnp.where(sel, ux, 0.0))
        yi = jnp.sum(jnp.where(sel, uy, 0.0))
        zi = jnp.sum(jnp.where(sel, uz, 0.0))
        uvecs.append((xi, yi, zi))
        sim = jnp.abs(ux * xi + uy * yi + uz * zi)
        s = jnp.where(sim > _THRESH, -jnp.inf, s)

    lane8 = lax.broadcasted_iota(jnp.int32, (1, 8), 1)
    out = jnp.where(lane8 == 1, picked[1], picked[0])
    out = jnp.where(lane8 == 2, picked[2], out)
    idx8_ref[...] = out
    idx1_ref[...] = out[:, :_VPTS]

    lane3 = lax.broadcasted_iota(jnp.int32, (1, _VPTS), 1)
    for k in range(_VPTS):
        xi, yi, zi = uvecs[k]
        vp_ref[k:k + 1, :] = jnp.where(
            lane3 == 0, xi, jnp.where(lane3 == 1, yi, zi))


def _sc_gather_body(idx8_hbm, inds_hbm, xf_hbm, s1_hbm,
                    ind1_out, go_hbm,
                    idx8_v, rows8_v, idxc4_v, g_v, sem, wsem):
    wid = lax.axis_index("s")

    @pl.when(wid < _TILES_USED)
    def _():
        pltpu.sync_copy(idx8_hbm, idx8_v)
        pltpu.async_copy(inds_hbm.at[idx8_v.at[0]], rows8_v, sem).wait()

        @pl.when(wid == 0)
        def _():
            pltpu.sync_copy(rows8_v.at[pl.ds(0, _VPTS)], ind1_out)

        vpt = wid // (_NODES // _CHUNK)
        c0 = (wid % (_NODES // _CHUNK)) * _CHUNK
        for j in range(_CHUNK // 16):
            base = rows8_v[vpt, pl.ds(c0 + j * 16, 16)]
            idxc4_v[0, pl.ds(j * 16, 16)] = base
            idxc4_v[1, pl.ds(j * 16, 16)] = base + _SEG
            idxc4_v[2, pl.ds(j * 16, 16)] = base + 2 * _SEG
            idxc4_v[3, pl.ds(j * 16, 16)] = base
        copies = [
            pltpu.async_copy(xf_hbm.at[idxc4_v.at[c]], g_v.at[c], sem)
            for c in range(3)
        ]
        copies.append(
            pltpu.async_copy(s1_hbm.at[idxc4_v.at[3]], g_v.at[3], sem))
        for cp in copies:
            cp.wait()
        writes = [
            pltpu.async_copy(
                g_v.at[c],
                go_hbm.at[pl.ds(c * (_VPTS * _NODES) + wid * _CHUNK, _CHUNK)],
                wsem)
            for c in range(4)
        ]
        for w in writes:
            w.wait()


def _graph_body(g_ref, pred_ref, nb_ref):
    coli = lax.broadcasted_iota(jnp.int32, (_NODES, _NODES), 1)
    rowi = lax.broadcasted_iota(jnp.int32, (_NODES, _NODES), 0)
    iden = (rowi == coli).astype(jnp.float32)
    ga = g_ref[...].reshape(1, 4 * _VPTS * _NODES)
    a_rows = []
    sc_rows = []
    for v in range(_VPTS):
        def comp(c):
            base = c * _VPTS * _NODES + v * _NODES
            return ga[:, base:base + _NODES]
        x, y, z, s = comp(0), comp(1), comp(2), comp(3)
        n = jnp.sqrt(x * x + y * y + z * z) + 1e-12
        u = jnp.concatenate([x / n, y / n, z / n], axis=0)
        dis = lax.dot_general(u, u, (((0,), (0,)), ((), ())),
                              preferred_element_type=jnp.float32)
        a_rows.append(jnp.abs(dis))
        sc_rows.append(lax.dot_general(iden, s, (((1,), (1,)), ((), ())),
                                       preferred_element_type=jnp.float32))
    a = jnp.concatenate(a_rows, axis=0)
    sc = jnp.concatenate(sc_rows, axis=0)
    colf = lax.broadcasted_iota(
        jnp.int32, (_VPTS * _NODES, _NODES), 1).astype(jnp.float32)
    vals_cols = []
    nb_cols = []
    for _ in range(_NEIGH):
        m = jnp.max(a, axis=1, keepdims=True)
        nbf = jnp.min(jnp.where(a == m, colf, jnp.float32(_NODES)),
                      axis=1, keepdims=True)
        vals_cols.append(m)
        nb_cols.append(nbf)
        a = jnp.where(colf == nbf, -jnp.inf, a)
    vals = jnp.concatenate(vals_cols, axis=1)
    nbs = jnp.concatenate(nb_cols, axis=1).astype(jnp.int32)
    pred_all = vals + 0.1 * sc
    for v in range(_VPTS):
        sl = slice(v * _NODES, (v + 1) * _NODES)
        pred_ref[v] = pred_all[sl, :]
        nb_ref[v] = nbs[sl, :]


def kernel(scores, xyz, ind1_scale):
    f32 = jnp.float32
    i32 = jnp.int32

    st = jnp.concatenate([scores[None, :], xyz.T], axis=0)
    idx8, idx1, vpts1, xf = pl.pallas_call(
        _nms_body,
        out_shape=(jax.ShapeDtypeStruct((1, 8), i32),
                   jax.ShapeDtypeStruct((1, _VPTS), i32),
                   jax.ShapeDtypeStruct((_VPTS, _VPTS), f32),
                   jax.ShapeDtypeStruct((3 * _SEG,), f32)),
    )(st)

    sc_fn = pl.kernel(
        _sc_gather_body,
        out_type=(jax.ShapeDtypeStruct((_VPTS, _NODES), i32),
                  jax.ShapeDtypeStruct((4 * _VPTS * _NODES,), f32)),
        mesh=plsc.VectorSubcoreMesh(core_axis_name="c", subcore_axis_name="s",
                                    num_cores=1),
        scratch_types=[
            pltpu.VMEM((1, 8), i32),
            pltpu.VMEM((8, _NODES), i32),
            pltpu.VMEM((4, _CHUNK), i32),
            pltpu.VMEM((4, _CHUNK), f32),
            pltpu.SemaphoreType.DMA,
            pltpu.SemaphoreType.DMA,
        ],
    )
    ind1, go = sc_fn(idx8, ind1_scale, xf, scores)

    pred, nbs = pl.pallas_call(
        _graph_body,
        out_shape=(jax.ShapeDtypeStruct((_VPTS, _NODES, _NEIGH), f32),
                   jax.ShapeDtypeStruct((_VPTS, _NODES, _NEIGH), i32)),
    )(go)

    center = jnp.broadcast_to(
        jnp.repeat(jnp.arange(_NODES, dtype=i32), _NEIGH)[None, :],
        (_VPTS, _NODES * _NEIGH))
    edge1 = jnp.stack([center, nbs.reshape(_VPTS, -1)], axis=1)
    return pred, idx1.reshape(_VPTS), ind1, edge1, vpts1
